# baseline (speedup 1.0000x reference)
import functools

import jax
import jax.numpy as jnp
from jax import lax
from jax.experimental import pallas as pl
from jax.experimental.pallas import tpu as pltpu
from jax.experimental.pallas import tpu_sc as plsc

B = 4096
D = 64
V = 100000
S = 4096
N_IDS = B + S
BROWS = (V + 127) // 128

_NC = 2
_NS = 16
_NW = _NC * _NS
_PER_W = N_IDS // _NW
_CHUNK = 128
_NCHUNK = _PER_W // _CHUNK


def _sc_gather_body(widx_hbm, bidx_hbm, w_hbm, b_hbm, out_w, out_b,
                    widx_v, bidx_v, wrows_v, brows_v, sem):
    wid = lax.axis_index("s") * _NC + lax.axis_index("c")
    base = wid * _PER_W
    pltpu.sync_copy(widx_hbm.at[wid], widx_v)
    pltpu.sync_copy(bidx_hbm.at[wid], bidx_v)
    for j in range(_NCHUNK):
        cw = pltpu.async_copy(w_hbm.at[widx_v.at[j]], wrows_v.at[j], sem)
        cb = pltpu.async_copy(b_hbm.at[bidx_v.at[j]], brows_v.at[j], sem)
        cw.wait()
        cb.wait()
    for j in range(_NCHUNK):
        pltpu.sync_copy(wrows_v.at[j],
                        out_w.at[pl.ds(base + j * _CHUNK, _CHUNK)])
        pltpu.sync_copy(brows_v.at[j],
                        out_b.at[pl.ds(base + j * _CHUNK, _CHUNK)])


@jax.jit
def _sc_gather(widx, bidx, w2, bpad):
    mesh = plsc.VectorSubcoreMesh(core_axis_name="c", subcore_axis_name="s")
    return pl.kernel(
        _sc_gather_body,
        out_type=(
            jax.ShapeDtypeStruct((N_IDS, 128), jnp.float32),
            jax.ShapeDtypeStruct((N_IDS, 128), jnp.float32),
        ),
        mesh=mesh,
        scratch_types=[
            pltpu.VMEM((_NCHUNK, _CHUNK), jnp.int32),
            pltpu.VMEM((_NCHUNK, _CHUNK), jnp.int32),
            pltpu.VMEM((_NCHUNK, _CHUNK, 128), jnp.float32),
            pltpu.VMEM((_NCHUNK, _CHUNK, 128), jnp.float32),
            pltpu.SemaphoreType.DMA,
        ],
    )(widx, bidx, w2, bpad)


_TB = 256
_GRID = B // _TB
_SCALE = 1.0 / (B * (S + 1))
_EPS = 1e-12


def _half_mask(ids):
    lane = lax.broadcasted_iota(jnp.int32, (ids.shape[0], 128), 1)
    par = ids & 1
    left = (lane < 64) & (par == 0)
    right = (lane >= 64) & (par == 1)
    return (left | right).astype(jnp.float32)


def _bias_onehot(ids):
    lane = lax.broadcasted_iota(jnp.int32, (ids.shape[0], 128), 1)
    return (lane == (ids & 127)).astype(jnp.float32)


def _tc_body(x_ref, twr_ref, tbr_ref, tid_ref, tec_ref,
             swr_ref, sbr_ref, sid_ref, sec_ref, out_ref, rhs_ref):
    i = pl.program_id(0)

    @pl.when(i == 0)
    def _build_rhs():
        sid = sid_ref[...]
        rhs_ref[:, 0:128] = swr_ref[...] * _half_mask(sid)
        lane = lax.broadcasted_iota(jnp.int32, (S, 128), 1)
        first = (lane == 0).astype(jnp.float32)
        rhs_ref[:, 128:256] = (sbr_ref[...] * _bias_onehot(sid)
                               - first * jnp.log(sec_ref[...]))

    x = x_ref[...]
    xa = jnp.concatenate(
        [x, x, jnp.ones((_TB, 128), jnp.float32)], axis=1)
    logits = lax.dot_general(
        xa, rhs_ref[...], (((1,), (1,)), ((), ())),
        preferred_element_type=jnp.float32)
    p = jax.nn.sigmoid(logits)
    part = jnp.sum(-jnp.log(jnp.clip(1.0 - p, _EPS, 1.0)))

    tid = tid_ref[...]
    xx = jnp.concatenate([x, x], axis=1)
    txw = jnp.sum(xx * (twr_ref[...] * _half_mask(tid)), axis=1,
                  keepdims=True)
    tb = jnp.sum(tbr_ref[...] * _bias_onehot(tid), axis=1, keepdims=True)
    tl = txw + tb - jnp.log(tec_ref[...])
    pt = jax.nn.sigmoid(tl)
    part += jnp.sum(-jnp.log(jnp.clip(pt, _EPS, 1.0)))

    @pl.when(i == 0)
    def _init():
        out_ref[0, 0] = 0.0

    out_ref[0, 0] += part * _SCALE


@functools.partial(jax.jit, static_argnames=("interpret",))
def _tc_loss(inputs, twr, tbr, tids, tec, swr, sbr, sids, sec,
             interpret=False):
    out = pl.pallas_call(
        _tc_body,
        grid=(_GRID,),
        in_specs=[
            pl.BlockSpec((_TB, D), lambda i: (i, 0)),
            pl.BlockSpec((_TB, 128), lambda i: (i, 0)),
            pl.BlockSpec((_TB, 128), lambda i: (i, 0)),
            pl.BlockSpec((_TB, 1), lambda i: (i, 0)),
            pl.BlockSpec((_TB, 1), lambda i: (i, 0)),
            pl.BlockSpec((S, 128), lambda i: (0, 0)),
            pl.BlockSpec((S, 128), lambda i: (0, 0)),
            pl.BlockSpec((S, 1), lambda i: (0, 0)),
            pl.BlockSpec((S, 1), lambda i: (0, 0)),
        ],
        out_specs=pl.BlockSpec(memory_space=pltpu.SMEM),
        out_shape=jax.ShapeDtypeStruct((1, 1), jnp.float32),
        scratch_shapes=[pltpu.VMEM((S, 256), jnp.float32)],
        interpret=interpret,
    )(inputs, twr, tbr, tids, tec, swr, sbr, sids, sec)
    return out[0, 0]


def kernel(inputs, labels, weights, biases, sampled_candidates,
           true_expected_count, sampled_expected_count):
    ids = jnp.concatenate(
        [labels.reshape(-1).astype(jnp.int32),
         sampled_candidates.astype(jnp.int32)], axis=0)
    w2 = weights.reshape(V // 2, 128)
    bpad = jnp.pad(biases, (0, BROWS * 128 - V)).reshape(BROWS, 128)
    widx = (ids >> 1).reshape(_NW, _NCHUNK, _CHUNK)
    bidx = (ids >> 7).reshape(_NW, _NCHUNK, _CHUNK)
    wrows, brows = _sc_gather(widx, bidx, w2, bpad)
    return _tc_loss(inputs,
                    wrows[:B], brows[:B],
                    ids[:B].reshape(B, 1),
                    true_expected_count,
                    wrows[B:], brows[B:],
                    ids[B:].reshape(S, 1),
                    sampled_expected_count.reshape(S, 1))


# speedup vs baseline: 1.0019x; 1.0019x over previous
import functools

import jax
import jax.numpy as jnp
from jax import lax
from jax.experimental import pallas as pl
from jax.experimental.pallas import tpu as pltpu
from jax.experimental.pallas import tpu_sc as plsc

B = 4096
D = 64
V = 100000
S = 4096
N_IDS = B + S
BROWS = (V + 127) // 128

_NC = 2
_NS = 16
_NW = _NC * _NS
_PER_W = N_IDS // _NW
_CHUNK = 128
_NCHUNK = _PER_W // _CHUNK


def _sc_gather_body(widx_hbm, bidx_hbm, w_hbm, b_hbm, out_w, out_b,
                    widx_v, bidx_v, wrows_v, bval_v, sem):
    wid = lax.axis_index("s") * _NC + lax.axis_index("c")
    base = wid * _PER_W
    pltpu.sync_copy(widx_hbm.at[wid], widx_v)
    pltpu.sync_copy(bidx_hbm.at[wid], bidx_v)
    copies = []
    for j in range(_NCHUNK):
        copies.append(pltpu.async_copy(w_hbm.at[widx_v.at[j]],
                                       wrows_v.at[j], sem))
        copies.append(pltpu.async_copy(b_hbm.at[bidx_v.at[j]],
                                       bval_v.at[j], sem))
    for c in copies:
        c.wait()
    for j in range(_NCHUNK):
        pltpu.sync_copy(wrows_v.at[j],
                        out_w.at[pl.ds(base + j * _CHUNK, _CHUNK)])
        pltpu.sync_copy(bval_v.at[j],
                        out_b.at[pl.ds(base + j * _CHUNK, _CHUNK)])


@jax.jit
def _sc_gather(widx, bidx, w2, biases):
    mesh = plsc.VectorSubcoreMesh(core_axis_name="c", subcore_axis_name="s")
    return pl.kernel(
        _sc_gather_body,
        out_type=(
            jax.ShapeDtypeStruct((N_IDS, 128), jnp.float32),
            jax.ShapeDtypeStruct((N_IDS,), jnp.float32),
        ),
        mesh=mesh,
        scratch_types=[
            pltpu.VMEM((_NCHUNK, _CHUNK), jnp.int32),
            pltpu.VMEM((_NCHUNK, _CHUNK), jnp.int32),
            pltpu.VMEM((_NCHUNK, _CHUNK, 128), jnp.float32),
            pltpu.VMEM((_NCHUNK, _CHUNK), jnp.float32),
            pltpu.SemaphoreType.DMA,
        ],
    )(widx, bidx, w2, biases)


_TB = 256
_GRID = B // _TB
_SCALE = 1.0 / (B * (S + 1))
_EPS = 1e-12


def _tc_body(x_ref, xf_ref, twr_ref, tb_ref, tid_ref, tec_ref,
             swr_ref, sb_ref, sid_ref, sec_ref, out_ref, rhs_ref):
    i = pl.program_id(0)

    @pl.when(i == 0)
    def _prep():
        sid = sid_ref[...]
        wsel = jnp.where((sid & 1) == 0,
                         swr_ref[:, :64], swr_ref[:, 64:])
        rhs_ref[:, 0:64] = wsel
        bcol = sb_ref[...] - jnp.log(sec_ref[...])
        lane64 = lax.broadcasted_iota(jnp.int32, (S, 64), 1)
        rhs_ref[:, 64:128] = jnp.where(lane64 == 0, bcol, 0.0)
        tid = tid_ref[...]
        twsel = jnp.where((tid & 1) == 0,
                          twr_ref[:, :64], twr_ref[:, 64:])
        txw = jnp.sum(xf_ref[...] * twsel, axis=1, keepdims=True)
        tl = txw + tb_ref[...] - jnp.log(tec_ref[...])
        pt = jax.nn.sigmoid(tl)
        tsum = jnp.sum(-jnp.log(jnp.clip(pt, _EPS, 1.0)))
        out_ref[0, 0] = tsum * _SCALE

    x = x_ref[...]
    xa = jnp.concatenate(
        [x, jnp.ones((_TB, 64), jnp.float32)], axis=1)
    logits = lax.dot_general(
        xa, rhs_ref[...], (((1,), (1,)), ((), ())),
        preferred_element_type=jnp.float32)
    p = jax.nn.sigmoid(logits)
    part = jnp.sum(-jnp.log(jnp.clip(1.0 - p, _EPS, 1.0)))
    out_ref[0, 0] += part * _SCALE


@functools.partial(jax.jit, static_argnames=("interpret",))
def _tc_loss(inputs, twr, tb, tids, tec, swr, sb, sids, sec,
             interpret=False):
    out = pl.pallas_call(
        _tc_body,
        grid=(_GRID,),
        in_specs=[
            pl.BlockSpec((_TB, D), lambda i: (i, 0)),
            pl.BlockSpec((B, D), lambda i: (0, 0)),
            pl.BlockSpec((B, 128), lambda i: (0, 0)),
            pl.BlockSpec((B, 1), lambda i: (0, 0)),
            pl.BlockSpec((B, 1), lambda i: (0, 0)),
            pl.BlockSpec((B, 1), lambda i: (0, 0)),
            pl.BlockSpec((S, 128), lambda i: (0, 0)),
            pl.BlockSpec((S, 1), lambda i: (0, 0)),
            pl.BlockSpec((S, 1), lambda i: (0, 0)),
            pl.BlockSpec((S, 1), lambda i: (0, 0)),
        ],
        out_specs=pl.BlockSpec(memory_space=pltpu.SMEM),
        out_shape=jax.ShapeDtypeStruct((1, 1), jnp.float32),
        scratch_shapes=[pltpu.VMEM((S, 128), jnp.float32)],
        interpret=interpret,
    )(inputs, inputs, twr, tb, tids, tec, swr, sb, sids, sec)
    return out[0, 0]


def kernel(inputs, labels, weights, biases, sampled_candidates,
           true_expected_count, sampled_expected_count):
    ids = jnp.concatenate(
        [labels.reshape(-1).astype(jnp.int32),
         sampled_candidates.astype(jnp.int32)], axis=0)
    w2 = weights.reshape(V // 2, 128)
    ids3 = ids.reshape(_NW, _NCHUNK, _CHUNK)
    wrows, bvals = _sc_gather(ids3 >> 1, ids3, w2, biases)
    return _tc_loss(inputs,
                    wrows[:B], bvals[:B].reshape(B, 1),
                    ids[:B].reshape(B, 1),
                    true_expected_count,
                    wrows[B:], bvals[B:].reshape(S, 1),
                    ids[B:].reshape(S, 1),
                    sampled_expected_count.reshape(S, 1))


# speedup vs baseline: 1.0539x; 1.0519x over previous
import functools

import jax
import jax.numpy as jnp
from jax import lax
from jax.experimental import pallas as pl
from jax.experimental.pallas import tpu as pltpu
from jax.experimental.pallas import tpu_sc as plsc

B = 4096
D = 64
V = 100000
S = 4096
N_IDS = B + S
BROWS = (V + 127) // 128

_NC = 2
_NS = 16
_NW = _NC * _NS
_PER_W = N_IDS // _NW
_CHUNK = 128
_NCHUNK = _PER_W // _CHUNK


def _sc_gather_body(idx_hbm, w_hbm, b_hbm, out_w, out_b,
                    idx_v, wrows_v, bval_v, sem):
    wid = lax.axis_index("s") * _NC + lax.axis_index("c")
    base = wid * _PER_W
    pltpu.sync_copy(idx_hbm.at[wid], idx_v)
    copies = []
    for j in range(_NCHUNK):
        copies.append(pltpu.async_copy(w_hbm.at[idx_v.at[j]],
                                       wrows_v.at[j], sem))
        copies.append(pltpu.async_copy(b_hbm.at[idx_v.at[j]],
                                       bval_v.at[j], sem))
    for c in copies:
        c.wait()
    for j in range(_NCHUNK):
        pltpu.sync_copy(wrows_v.at[j],
                        out_w.at[pl.ds(base + j * _CHUNK, _CHUNK)])
        pltpu.sync_copy(bval_v.at[j],
                        out_b.at[pl.ds(base + j * _CHUNK, _CHUNK)])


@jax.jit
def _sc_gather(idx, weights, biases):
    mesh = plsc.VectorSubcoreMesh(core_axis_name="c", subcore_axis_name="s")
    return pl.kernel(
        _sc_gather_body,
        out_type=(
            jax.ShapeDtypeStruct((N_IDS, D), jnp.float32),
            jax.ShapeDtypeStruct((N_IDS,), jnp.float32),
        ),
        mesh=mesh,
        compiler_params=pltpu.CompilerParams(use_tc_tiling_on_sc=False),
        scratch_types=[
            pltpu.VMEM((_NCHUNK, _CHUNK), jnp.int32),
            pltpu.VMEM((_NCHUNK, _CHUNK, D), jnp.float32),
            pltpu.VMEM((_NCHUNK, _CHUNK), jnp.float32),
            pltpu.SemaphoreType.DMA,
        ],
    )(idx, weights, biases)


_TB = 256
_GRID = B // _TB
_SCALE = 1.0 / (B * (S + 1))
_EPS = 1e-12


def _tc_body(x_ref, xf_ref, twr_ref, tb_ref, tec_ref,
             swr_ref, sb_ref, sec_ref, out_ref, rhs_ref):
    i = pl.program_id(0)

    @pl.when(i == 0)
    def _prep():
        rhs_ref[:, 0:D] = swr_ref[...]
        bcol = sb_ref[...] - jnp.log(sec_ref[...])
        lane64 = lax.broadcasted_iota(jnp.int32, (S, 64), 1)
        rhs_ref[:, 64:128] = jnp.where(lane64 == 0, bcol, 0.0)
        txw = jnp.sum(xf_ref[...] * twr_ref[...], axis=1, keepdims=True)
        tl = txw + tb_ref[...] - jnp.log(tec_ref[...])
        pt = jax.nn.sigmoid(tl)
        tsum = jnp.sum(-jnp.log(jnp.clip(pt, _EPS, 1.0)))
        out_ref[0, 0] = tsum * _SCALE

    x = x_ref[...]
    xa = jnp.concatenate(
        [x, jnp.ones((_TB, 64), jnp.float32)], axis=1)
    logits = lax.dot_general(
        xa, rhs_ref[...], (((1,), (1,)), ((), ())),
        preferred_element_type=jnp.float32)
    p = jax.nn.sigmoid(logits)
    part = jnp.sum(-jnp.log(jnp.clip(1.0 - p, _EPS, 1.0)))
    out_ref[0, 0] += part * _SCALE


@functools.partial(jax.jit, static_argnames=("interpret",))
def _tc_loss(inputs, twr, tb, tec, swr, sb, sec, interpret=False):
    out = pl.pallas_call(
        _tc_body,
        grid=(_GRID,),
        in_specs=[
            pl.BlockSpec((_TB, D), lambda i: (i, 0)),
            pl.BlockSpec((B, D), lambda i: (0, 0)),
            pl.BlockSpec((B, D), lambda i: (0, 0)),
            pl.BlockSpec((B, 1), lambda i: (0, 0)),
            pl.BlockSpec((B, 1), lambda i: (0, 0)),
            pl.BlockSpec((S, D), lambda i: (0, 0)),
            pl.BlockSpec((S, 1), lambda i: (0, 0)),
            pl.BlockSpec((S, 1), lambda i: (0, 0)),
        ],
        out_specs=pl.BlockSpec(memory_space=pltpu.SMEM),
        out_shape=jax.ShapeDtypeStruct((1, 1), jnp.float32),
        scratch_shapes=[pltpu.VMEM((S, 128), jnp.float32)],
        interpret=interpret,
    )(inputs, inputs, twr, tb, tec, swr, sb, sec)
    return out[0, 0]


def kernel(inputs, labels, weights, biases, sampled_candidates,
           true_expected_count, sampled_expected_count):
    ids = jnp.concatenate(
        [labels.reshape(-1).astype(jnp.int32),
         sampled_candidates.astype(jnp.int32)], axis=0)
    ids3 = ids.reshape(_NW, _NCHUNK, _CHUNK)
    wrows, bvals = _sc_gather(ids3, weights, biases)
    return _tc_loss(inputs,
                    wrows[:B], bvals[:B].reshape(B, 1),
                    true_expected_count,
                    wrows[B:], bvals[B:].reshape(S, 1),
                    sampled_expected_count.reshape(S, 1))
